# Initial kernel scaffold; baseline (speedup 1.0000x reference)
import jax
import jax.numpy as jnp
from jax.experimental import pallas as pl


def kernel(labels, losses):
    raise NotImplementedError("write your pallas kernel here")


# speedup vs baseline: 30.0056x; 30.0056x over previous
import functools

import jax
import jax.numpy as jnp
from jax import lax
from jax.experimental import pallas as pl
from jax.experimental.pallas import tpu as pltpu
from jax.experimental.pallas import tpu_sc as plsc

N = 262144
NT = 16
CH = N // NT
NCHUNK = CH // 16
NB = 2048
NBC = NB // 16
MAXPOS = 64
BATCH = 256


def _ohem_body(labels_hbm, keys_hbm, outp_hbm, outn_hbm,
               lab_v, key_v, outp_v, outn_v, histp_v, histn_v,
               rows_v, merg_v, shrows):
    wid = lax.axis_index("s")
    base = wid * CH
    pltpu.sync_copy(labels_hbm.at[pl.ds(base, CH)], lab_v)
    pltpu.sync_copy(keys_hbm.at[pl.ds(base, CH)], key_v)

    iota = lax.iota(jnp.int32, 16)
    ones = jnp.full((16,), 1, jnp.int32)
    zeros16 = jnp.full((16,), 0, jnp.int32)
    BIG = jnp.int32(1 << 30)

    def load_chunk(j):
        lab = lab_v[pl.ds(j * 16, 16)]
        key = key_v[pl.ds(j * 16, 16)]
        return lab, key

    def hist_round(up_shift, shift, prefp, prefn):
        def zbody(i, _):
            histp_v[pl.ds(i * 16, 16)] = zeros16
            histn_v[pl.ds(i * 16, 16)] = zeros16
            return jnp.int32(0)
        lax.fori_loop(0, NBC, zbody, jnp.int32(0))

        def body(j, _):
            lab, key = load_chunk(j)
            isp = lab == 1
            isn = lab == 0
            if up_shift < 32:
                pm = key >> up_shift
                isp = isp & (pm == prefp)
                isn = isn & (pm == prefn)
            d = (key >> shift) & (NB - 1)
            plsc.addupdate_scatter(histp_v, [d], ones, mask=isp)
            plsc.addupdate_scatter(histn_v, [d], ones, mask=isn)
            return jnp.int32(0)
        lax.fori_loop(0, NCHUNK, body, jnp.int32(0))

        pltpu.sync_copy(histp_v, shrows.at[0, wid])
        pltpu.sync_copy(histn_v, shrows.at[1, wid])
        plsc.subcore_barrier()

    def build_merged(c):
        pltpu.sync_copy(shrows.at[c], rows_v)

        def body(i, _):
            acc = zeros16
            for t in range(NT):
                acc = acc + rows_v[t, pl.ds(i * 16, 16)]
            merg_v[pl.ds(i * 16, 16)] = acc
            return jnp.int32(0)
        lax.fori_loop(0, NBC, body, jnp.int32(0))

    def merged_total():
        def body(i, acc):
            return acc + jnp.sum(merg_v[pl.ds(i * 16, 16)])
        return lax.fori_loop(0, NBC, body, jnp.int32(0))

    def dscan(K_rem):
        def body(i, carry):
            running, D, G = carry
            ci = NBC - 1 - i
            h = merg_v[pl.ds(ci * 16, 16)]
            s_in = lax.rev(jnp.cumsum(lax.rev(h, (0,))), (0,))
            S = s_in + running
            b_idx = ci * 16 + iota
            qual = S >= K_rem
            D = jnp.maximum(D, jnp.max(jnp.where(qual, b_idx, -1)))
            G = jnp.minimum(G, jnp.min(jnp.where(qual, S - h, BIG)))
            running = running + jnp.sum(h)
            return running, D, G
        _, D, G = lax.fori_loop(0, NBC, body,
                                (jnp.int32(0), jnp.int32(-1), BIG))
        return D, G

    def eq_base(D3):
        col = jnp.maximum(D3, 0) + zeros16
        vals = plsc.load_gather(rows_v, [iota, col])
        return jnp.sum(jnp.where(iota < wid, vals, jnp.int32(0)))

    hist_round(32, 21, jnp.int32(0), jnp.int32(0))

    build_merged(0)
    num_pos = merged_total()
    K_pos = jnp.minimum(num_pos, jnp.int32(MAXPOS))
    D1p, G = dscan(K_pos)
    Kp_rem = K_pos - G

    build_merged(1)
    num_neg = merged_total()
    K_neg = jnp.minimum(jnp.int32(BATCH) - K_pos, num_neg)
    D1n, G = dscan(K_neg)
    Kn_rem = K_neg - G
    plsc.subcore_barrier()

    hist_round(21, 10, D1p, D1n)
    build_merged(0)
    D2p, G = dscan(Kp_rem)
    Kp_rem = Kp_rem - G
    build_merged(1)
    D2n, G = dscan(Kn_rem)
    Kn_rem = Kn_rem - G
    plsc.subcore_barrier()

    pref2p = (D1p << 11) | D2p
    pref2n = (D1n << 11) | D2n
    hist_round(10, 0, pref2p, pref2n)
    build_merged(0)
    D3p, G = dscan(Kp_rem)
    r_p = Kp_rem - G
    base_p = eq_base(D3p)
    build_merged(1)
    D3n, G = dscan(Kn_rem)
    r_n = Kn_rem - G
    base_n = eq_base(D3n)

    T_p = (D1p << 21) | (D2p << 10) | D3p
    T_n = (D1n << 21) | (D2n << 10) | D3n
    maxkey = jnp.int32(0x7FFFFFFF)
    T_p = jnp.where(K_pos == 0, maxkey, T_p)
    r_p = jnp.where(K_pos == 0, jnp.int32(0), r_p)
    T_n = jnp.where(K_neg == 0, maxkey, T_n)
    r_n = jnp.where(K_neg == 0, jnp.int32(0), r_n)

    def body(j, carry):
        rkp, rkn = carry
        lab, key = load_chunk(j)
        isp = lab == 1
        isn = lab == 0

        eqp = isp & (key == T_p)
        eqpi = jnp.where(eqp, jnp.int32(1), jnp.int32(0))
        exclp = jnp.cumsum(eqpi) - eqpi
        selp = (isp & (key > T_p)) | (eqp & ((rkp + exclp) < r_p))
        outp_v[pl.ds(j * 16, 16)] = jnp.where(selp, jnp.int32(1), jnp.int32(0))
        rkp = rkp + jnp.sum(eqpi)

        eqn = isn & (key == T_n)
        eqni = jnp.where(eqn, jnp.int32(1), jnp.int32(0))
        excln = jnp.cumsum(eqni) - eqni
        seln = (isn & (key > T_n)) | (eqn & ((rkn + excln) < r_n))
        outn_v[pl.ds(j * 16, 16)] = jnp.where(seln, jnp.int32(1), jnp.int32(0))
        rkn = rkn + jnp.sum(eqni)
        return rkp, rkn

    lax.fori_loop(0, NCHUNK, body, (base_p, base_n))

    pltpu.sync_copy(outp_v, outp_hbm.at[pl.ds(base, CH)])
    pltpu.sync_copy(outn_v, outn_hbm.at[pl.ds(base, CH)])


_mesh = plsc.VectorSubcoreMesh(core_axis_name="c", subcore_axis_name="s",
                               num_cores=1)

_ohem = functools.partial(
    pl.kernel,
    mesh=_mesh,
    compiler_params=pltpu.CompilerParams(needs_layout_passes=False),
    out_type=[jax.ShapeDtypeStruct((N,), jnp.int32),
              jax.ShapeDtypeStruct((N,), jnp.int32)],
    scratch_types=[
        pltpu.VMEM((CH,), jnp.int32),
        pltpu.VMEM((CH,), jnp.int32),
        pltpu.VMEM((CH,), jnp.int32),
        pltpu.VMEM((CH,), jnp.int32),
        pltpu.VMEM((NB,), jnp.int32),
        pltpu.VMEM((NB,), jnp.int32),
        pltpu.VMEM((NT, NB), jnp.int32),
        pltpu.VMEM((NB,), jnp.int32),
        pltpu.VMEM_SHARED((2, NT, NB), jnp.int32),
    ],
)(_ohem_body)


def kernel(labels, losses):
    labels = labels.astype(jnp.int32)
    keys = jax.lax.bitcast_convert_type(losses.astype(jnp.float32), jnp.int32)
    outp, outn = _ohem(labels, keys)
    return outp != 0, outn != 0


# speedup vs baseline: 52.7368x; 1.7576x over previous
import functools

import jax
import jax.numpy as jnp
from jax import lax
from jax.experimental import pallas as pl
from jax.experimental.pallas import tpu as pltpu
from jax.experimental.pallas import tpu_sc as plsc

N = 262144
NT = 16
CH = N // NT
NCHUNK = CH // 16
NB = 2048
NBC = NB // 16
SL = NB // NT
MAXPOS = 64
BATCH = 256


def _ohem_body(labels_hbm, keys_hbm, outp_hbm, outn_hbm,
               lab_v, key_v, outp_v, outn_v, histp_v, histn_v,
               slice_v, mslice_v, merg_v, stats_v, statsme_v,
               shrows, shmerg, sstats):
    wid = lax.axis_index("s")
    base = wid * CH
    pltpu.sync_copy(labels_hbm.at[pl.ds(base, CH)], lab_v)
    pltpu.sync_copy(keys_hbm.at[pl.ds(base, CH)], key_v)

    iota = lax.iota(jnp.int32, 16)
    ones = jnp.full((16,), 1, jnp.int32)
    zeros16 = jnp.full((16,), 0, jnp.int32)
    BIG = jnp.int32(1 << 30)

    def load_chunk(j):
        return lab_v[pl.ds(j * 16, 16)], key_v[pl.ds(j * 16, 16)]

    def hist_pass(up_shift, shift, prefp, prefn):
        @plsc.parallel_loop(0, NBC, unroll=4)
        def _(i):
            histp_v[pl.ds(i * 16, 16)] = zeros16
            histn_v[pl.ds(i * 16, 16)] = zeros16

        @plsc.parallel_loop(0, NCHUNK, unroll=4)
        def _(j):
            lab, key = load_chunk(j)
            isp = lab == 1
            isn = lab == 0
            if up_shift < 32:
                pm = key >> up_shift
                isp = isp & (pm == prefp)
                isn = isn & (pm == prefn)
            d = (key >> shift) & (NB - 1)
            plsc.addupdate_scatter(histp_v, [d], ones, mask=isp)
            plsc.addupdate_scatter(histn_v, [d], ones, mask=isn)

        pltpu.sync_copy(histp_v, shrows.at[0, wid])
        pltpu.sync_copy(histn_v, shrows.at[1, wid])
        plsc.subcore_barrier()

    def build_merged(c):
        pltpu.sync_copy(shrows.at[c, :, pl.ds(wid * SL, SL)], slice_v)

        @plsc.parallel_loop(0, SL // 16, unroll=2)
        def _(i):
            acc = zeros16
            for t in range(NT):
                acc = acc + slice_v[t, pl.ds(i * 16, 16)]
            mslice_v[pl.ds(i * 16, 16)] = acc

        pltpu.sync_copy(mslice_v, shmerg.at[c, pl.ds(wid * SL, SL)])
        plsc.subcore_barrier()
        pltpu.sync_copy(shmerg.at[c], merg_v)

    def merged_total():
        def body(i, acc):
            return acc + jnp.sum(merg_v[pl.ds(i * 16, 16)])
        return lax.fori_loop(0, NBC, body, jnp.int32(0))

    def dscan(K_rem):
        def body(i, carry):
            running, D, G = carry
            ci = NBC - 1 - i
            h = merg_v[pl.ds(ci * 16, 16)]
            s_in = lax.rev(jnp.cumsum(lax.rev(h, (0,))), (0,))
            S = s_in + running
            b_idx = ci * 16 + iota
            qual = S >= K_rem
            D = jnp.maximum(D, jnp.max(jnp.where(qual, b_idx, -1)))
            G = jnp.minimum(G, jnp.min(jnp.where(qual, S - h, BIG)))
            running = running + jnp.sum(h)
            return running, D, G
        _, D, G = lax.fori_loop(0, NBC, body,
                                (jnp.int32(0), jnp.int32(-1), BIG))
        return D, G

    hist_pass(32, 21, jnp.int32(0), jnp.int32(0))

    build_merged(0)
    num_pos = merged_total()
    K_pos = jnp.minimum(num_pos, jnp.int32(MAXPOS))
    D1p, G = dscan(K_pos)
    Kp_rem = K_pos - G

    build_merged(1)
    num_neg = merged_total()
    K_neg = jnp.minimum(jnp.int32(BATCH) - K_pos, num_neg)
    D1n, G = dscan(K_neg)
    Kn_rem = K_neg - G

    hist_pass(21, 10, D1p, D1n)
    build_merged(0)
    D2p, G = dscan(Kp_rem)
    Kp_rem = Kp_rem - G
    build_merged(1)
    D2n, G = dscan(Kn_rem)
    Kn_rem = Kn_rem - G

    pref2p = (D1p << 11) | D2p
    pref2n = (D1n << 11) | D2n
    hist_pass(10, 0, pref2p, pref2n)
    build_merged(0)
    D3p, G = dscan(Kp_rem)
    r_p = Kp_rem - G
    build_merged(1)
    D3n, G = dscan(Kn_rem)
    r_n = Kn_rem - G

    e_p = jnp.max(plsc.load_gather(histp_v, [jnp.maximum(D3p, 0) + zeros16]))
    e_n = jnp.max(plsc.load_gather(histn_v, [jnp.maximum(D3n, 0) + zeros16]))
    statsme_v[...] = jnp.where(iota == 0, e_p,
                               jnp.where(iota == 1, e_n, jnp.int32(0)))
    pltpu.sync_copy(statsme_v, sstats.at[wid])
    plsc.subcore_barrier()
    pltpu.sync_copy(sstats, stats_v)
    colp = plsc.load_gather(stats_v, [iota, zeros16])
    coln = plsc.load_gather(stats_v, [iota, zeros16 + 1])
    below = iota < wid
    base_p = jnp.sum(jnp.where(below, colp, jnp.int32(0)))
    base_n = jnp.sum(jnp.where(below, coln, jnp.int32(0)))

    T_p = (D1p << 21) | (D2p << 10) | D3p
    T_n = (D1n << 21) | (D2n << 10) | D3n
    maxkey = jnp.int32(0x7FFFFFFF)
    T_p = jnp.where(K_pos == 0, maxkey, T_p)
    r_p = jnp.where(K_pos == 0, jnp.int32(0), r_p)
    T_n = jnp.where(K_neg == 0, maxkey, T_n)
    r_n = jnp.where(K_neg == 0, jnp.int32(0), r_n)

    allp = (base_p + e_p) <= r_p
    alln = (base_n + e_n) <= r_n
    partp = (base_p < r_p) & jnp.logical_not(allp)
    partn = (base_n < r_n) & jnp.logical_not(alln)

    @plsc.parallel_loop(0, NCHUNK, unroll=2)
    def _(j):
        lab, key = load_chunk(j)
        selp = (lab == 1) & ((key > T_p) | ((key == T_p) & allp))
        seln = (lab == 0) & ((key > T_n) | ((key == T_n) & alln))
        outp_v[pl.ds(j * 16, 16)] = jnp.where(selp, jnp.int32(1), jnp.int32(0))
        outn_v[pl.ds(j * 16, 16)] = jnp.where(seln, jnp.int32(1), jnp.int32(0))

    @pl.when(partp)
    def _():
        def fb(j, rk):
            lab, key = load_chunk(j)
            eq = (lab == 1) & (key == T_p)
            eqi = jnp.where(eq, jnp.int32(1), jnp.int32(0))
            excl = jnp.cumsum(eqi) - eqi
            add = eq & ((rk + excl) < r_p)
            cur = outp_v[pl.ds(j * 16, 16)]
            outp_v[pl.ds(j * 16, 16)] = cur | jnp.where(add, jnp.int32(1),
                                                        jnp.int32(0))
            return rk + jnp.sum(eqi)
        lax.fori_loop(0, NCHUNK, fb, base_p)

    @pl.when(partn)
    def _():
        def fb(j, rk):
            lab, key = load_chunk(j)
            eq = (lab == 0) & (key == T_n)
            eqi = jnp.where(eq, jnp.int32(1), jnp.int32(0))
            excl = jnp.cumsum(eqi) - eqi
            add = eq & ((rk + excl) < r_n)
            cur = outn_v[pl.ds(j * 16, 16)]
            outn_v[pl.ds(j * 16, 16)] = cur | jnp.where(add, jnp.int32(1),
                                                        jnp.int32(0))
            return rk + jnp.sum(eqi)
        lax.fori_loop(0, NCHUNK, fb, base_n)

    pltpu.sync_copy(outp_v, outp_hbm.at[pl.ds(base, CH)])
    pltpu.sync_copy(outn_v, outn_hbm.at[pl.ds(base, CH)])


_mesh = plsc.VectorSubcoreMesh(core_axis_name="c", subcore_axis_name="s",
                               num_cores=1)

_ohem = functools.partial(
    pl.kernel,
    mesh=_mesh,
    compiler_params=pltpu.CompilerParams(needs_layout_passes=False),
    out_type=[jax.ShapeDtypeStruct((N,), jnp.int32),
              jax.ShapeDtypeStruct((N,), jnp.int32)],
    scratch_types=[
        pltpu.VMEM((CH,), jnp.int32),
        pltpu.VMEM((CH,), jnp.int32),
        pltpu.VMEM((CH,), jnp.int32),
        pltpu.VMEM((CH,), jnp.int32),
        pltpu.VMEM((NB,), jnp.int32),
        pltpu.VMEM((NB,), jnp.int32),
        pltpu.VMEM((NT, SL), jnp.int32),
        pltpu.VMEM((SL,), jnp.int32),
        pltpu.VMEM((NB,), jnp.int32),
        pltpu.VMEM((NT, 16), jnp.int32),
        pltpu.VMEM((16,), jnp.int32),
        pltpu.VMEM_SHARED((2, NT, NB), jnp.int32),
        pltpu.VMEM_SHARED((2, NB), jnp.int32),
        pltpu.VMEM_SHARED((NT, 16), jnp.int32),
    ],
)(_ohem_body)


def kernel(labels, losses):
    labels = labels.astype(jnp.int32)
    keys = jax.lax.bitcast_convert_type(losses.astype(jnp.float32), jnp.int32)
    outp, outn = _ohem(labels, keys)
    return outp != 0, outn != 0


# speedup vs baseline: 55.4371x; 1.0512x over previous
import functools

import jax
import jax.numpy as jnp
from jax import lax
from jax.experimental import pallas as pl
from jax.experimental.pallas import tpu as pltpu
from jax.experimental.pallas import tpu_sc as plsc

N = 262144
NT = 16
CH = N // NT
NCHUNK = CH // 16
NB = 2048
NB2 = 2 * NB
SL = NB2 // NT
NBC = NB // 16
MAXPOS = 64
BATCH = 256


def _ohem_body(labels_hbm, losses_hbm, out_hbm,
               lab_v, loss_v, out_v, hist_v,
               slice_v, mslice_v, merg_v, stats_v, statsme_v,
               shrows, shmerg, sstats):
    wid = lax.axis_index("s")
    base = wid * CH
    pltpu.sync_copy(labels_hbm.at[pl.ds(base, CH)], lab_v)
    pltpu.sync_copy(losses_hbm.at[pl.ds(base, CH)], loss_v)

    iota = lax.iota(jnp.int32, 16)
    ones = jnp.full((16,), 1, jnp.int32)
    zeros16 = jnp.full((16,), 0, jnp.int32)
    BIG = jnp.int32(1 << 30)

    def load_chunk(j):
        lab = lab_v[pl.ds(j * 16, 16)]
        key = plsc.bitcast(loss_v[pl.ds(j * 16, 16)], jnp.int32)
        return lab, key

    def hist_pass(up_shift, shift, prefp, prefn):
        @plsc.parallel_loop(0, NB2 // 16, unroll=4)
        def _(i):
            hist_v[pl.ds(i * 16, 16)] = zeros16

        @plsc.parallel_loop(0, NCHUNK, unroll=4)
        def _(j):
            lab, key = load_chunk(j)
            isp = lab == 1
            isn = lab == 0
            if up_shift < 32:
                pm = key >> up_shift
                isp = isp & (pm == prefp)
                isn = isn & (pm == prefn)
            d = ((key >> shift) & (NB - 1)) + jnp.where(isn, jnp.int32(NB),
                                                        jnp.int32(0))
            plsc.addupdate_scatter(hist_v, [d], ones, mask=isp | isn)

        pltpu.sync_copy(hist_v, shrows.at[wid])
        plsc.subcore_barrier()

    def build_merged():
        pltpu.sync_copy(shrows.at[:, pl.ds(wid * SL, SL)], slice_v)

        @plsc.parallel_loop(0, SL // 16, unroll=2)
        def _(i):
            acc = zeros16
            for t in range(NT):
                acc = acc + slice_v[t, pl.ds(i * 16, 16)]
            mslice_v[pl.ds(i * 16, 16)] = acc

        pltpu.sync_copy(mslice_v, shmerg.at[pl.ds(wid * SL, SL)])
        plsc.subcore_barrier()
        pltpu.sync_copy(shmerg, merg_v)

    def merged_total(off):
        def body(i, acc):
            return acc + jnp.sum(merg_v[pl.ds(off + i * 16, 16)])
        return lax.fori_loop(0, NBC, body, jnp.int32(0))

    def dscan(off, K_rem):
        def body(i, carry):
            running, D, G = carry
            ci = NBC - 1 - i
            h = merg_v[pl.ds(off + ci * 16, 16)]
            s_in = lax.rev(jnp.cumsum(lax.rev(h, (0,))), (0,))
            S = s_in + running
            b_idx = ci * 16 + iota
            qual = S >= K_rem
            D = jnp.maximum(D, jnp.max(jnp.where(qual, b_idx, -1)))
            G = jnp.minimum(G, jnp.min(jnp.where(qual, S - h, BIG)))
            running = running + jnp.sum(h)
            return running, D, G
        _, D, G = lax.fori_loop(0, NBC, body,
                                (jnp.int32(0), jnp.int32(-1), BIG))
        return D, G

    hist_pass(32, 21, jnp.int32(0), jnp.int32(0))
    build_merged()

    num_pos = merged_total(0)
    K_pos = jnp.minimum(num_pos, jnp.int32(MAXPOS))
    D1p, G = dscan(0, K_pos)
    Kp_rem = K_pos - G

    num_neg = merged_total(NB)
    K_neg = jnp.minimum(jnp.int32(BATCH) - K_pos, num_neg)
    D1n, G = dscan(NB, K_neg)
    Kn_rem = K_neg - G

    hist_pass(21, 10, D1p, D1n)
    build_merged()
    D2p, G = dscan(0, Kp_rem)
    Kp_rem = Kp_rem - G
    D2n, G = dscan(NB, Kn_rem)
    Kn_rem = Kn_rem - G

    pref2p = (D1p << 11) | D2p
    pref2n = (D1n << 11) | D2n
    hist_pass(10, 0, pref2p, pref2n)
    build_merged()
    D3p, G = dscan(0, Kp_rem)
    r_p = Kp_rem - G
    D3n, G = dscan(NB, Kn_rem)
    r_n = Kn_rem - G

    e_p = jnp.max(plsc.load_gather(hist_v, [jnp.maximum(D3p, 0) + zeros16]))
    e_n = jnp.max(plsc.load_gather(
        hist_v, [jnp.maximum(D3n, 0) + NB + zeros16]))
    statsme_v[...] = jnp.where(iota == 0, e_p,
                               jnp.where(iota == 1, e_n, jnp.int32(0)))
    pltpu.sync_copy(statsme_v, sstats.at[wid])
    plsc.subcore_barrier()
    pltpu.sync_copy(sstats, stats_v)
    colp = plsc.load_gather(stats_v, [iota, zeros16])
    coln = plsc.load_gather(stats_v, [iota, zeros16 + 1])
    below = iota < wid
    base_p = jnp.sum(jnp.where(below, colp, jnp.int32(0)))
    base_n = jnp.sum(jnp.where(below, coln, jnp.int32(0)))

    T_p = (D1p << 21) | (D2p << 10) | D3p
    T_n = (D1n << 21) | (D2n << 10) | D3n
    maxkey = jnp.int32(0x7FFFFFFF)
    T_p = jnp.where(K_pos == 0, maxkey, T_p)
    r_p = jnp.where(K_pos == 0, jnp.int32(0), r_p)
    T_n = jnp.where(K_neg == 0, maxkey, T_n)
    r_n = jnp.where(K_neg == 0, jnp.int32(0), r_n)

    allp = (base_p + e_p) <= r_p
    alln = (base_n + e_n) <= r_n
    partp = (base_p < r_p) & jnp.logical_not(allp)
    partn = (base_n < r_n) & jnp.logical_not(alln)

    @plsc.parallel_loop(0, NCHUNK, unroll=2)
    def _(j):
        lab, key = load_chunk(j)
        selp = (lab == 1) & ((key > T_p) | ((key == T_p) & allp))
        seln = (lab == 0) & ((key > T_n) | ((key == T_n) & alln))
        out_v[pl.ds(j * 16, 16)] = (
            jnp.where(selp, jnp.int32(1), jnp.int32(0))
            | jnp.where(seln, jnp.int32(2), jnp.int32(0)))

    @pl.when(partp)
    def _():
        def fb(j, rk):
            lab, key = load_chunk(j)
            eq = (lab == 1) & (key == T_p)
            eqi = jnp.where(eq, jnp.int32(1), jnp.int32(0))
            excl = jnp.cumsum(eqi) - eqi
            add = eq & ((rk + excl) < r_p)
            cur = out_v[pl.ds(j * 16, 16)]
            out_v[pl.ds(j * 16, 16)] = cur | jnp.where(add, jnp.int32(1),
                                                       jnp.int32(0))
            return rk + jnp.sum(eqi)
        lax.fori_loop(0, NCHUNK, fb, base_p)

    @pl.when(partn)
    def _():
        def fb(j, rk):
            lab, key = load_chunk(j)
            eq = (lab == 0) & (key == T_n)
            eqi = jnp.where(eq, jnp.int32(1), jnp.int32(0))
            excl = jnp.cumsum(eqi) - eqi
            add = eq & ((rk + excl) < r_n)
            cur = out_v[pl.ds(j * 16, 16)]
            out_v[pl.ds(j * 16, 16)] = cur | jnp.where(add, jnp.int32(2),
                                                       jnp.int32(0))
            return rk + jnp.sum(eqi)
        lax.fori_loop(0, NCHUNK, fb, base_n)

    pltpu.sync_copy(out_v, out_hbm.at[pl.ds(base, CH)])


_mesh = plsc.VectorSubcoreMesh(core_axis_name="c", subcore_axis_name="s",
                               num_cores=1)

_ohem = functools.partial(
    pl.kernel,
    mesh=_mesh,
    compiler_params=pltpu.CompilerParams(needs_layout_passes=False),
    out_type=jax.ShapeDtypeStruct((N,), jnp.int32),
    scratch_types=[
        pltpu.VMEM((CH,), jnp.int32),
        pltpu.VMEM((CH,), jnp.float32),
        pltpu.VMEM((CH,), jnp.int32),
        pltpu.VMEM((NB2,), jnp.int32),
        pltpu.VMEM((NT, SL), jnp.int32),
        pltpu.VMEM((SL,), jnp.int32),
        pltpu.VMEM((NB2,), jnp.int32),
        pltpu.VMEM((NT, 16), jnp.int32),
        pltpu.VMEM((16,), jnp.int32),
        pltpu.VMEM_SHARED((NT, NB2), jnp.int32),
        pltpu.VMEM_SHARED((NB2,), jnp.int32),
        pltpu.VMEM_SHARED((NT, 16), jnp.int32),
    ],
)(_ohem_body)


def kernel(labels, losses):
    labels = labels.astype(jnp.int32)
    packed = _ohem(labels, losses.astype(jnp.float32))
    return (packed & 1) != 0, (packed & 2) != 0


# speedup vs baseline: 60.0094x; 1.0825x over previous
import functools

import jax
import jax.numpy as jnp
from jax import lax
from jax.experimental import pallas as pl
from jax.experimental.pallas import tpu as pltpu
from jax.experimental.pallas import tpu_sc as plsc

N = 262144
NT = 16
CH = N // NT
NCHUNK = CH // 16
NB = 2048
NB2 = 2 * NB
SL = NB2 // NT
NBC = NB // 16
MAXPOS = 64
BATCH = 256


def _ohem_body(labels_hbm, losses_hbm, out_hbm,
               lab_v, loss_v, out_v, hist_v,
               slice_v, mslice_v, csums_v, csall_v, tmp16_v,
               stats_v, statsme_v,
               shrows, shmerg, shcsum, sstats):
    wid = lax.axis_index("s")
    base = wid * CH
    pltpu.sync_copy(labels_hbm.at[pl.ds(base, CH)], lab_v)
    pltpu.sync_copy(losses_hbm.at[pl.ds(base, CH)], loss_v)

    iota = lax.iota(jnp.int32, 16)
    ones = jnp.full((16,), 1, jnp.int32)
    zeros16 = jnp.full((16,), 0, jnp.int32)
    BIG = jnp.int32(1 << 30)

    def load_chunk(j):
        lab = lab_v[pl.ds(j * 16, 16)]
        key = plsc.bitcast(loss_v[pl.ds(j * 16, 16)], jnp.int32)
        return lab, key

    def hist_pass(up_shift, shift, prefp, prefn):
        @plsc.parallel_loop(0, NB2 // 16, unroll=4)
        def _(i):
            hist_v[pl.ds(i * 16, 16)] = zeros16

        @plsc.parallel_loop(0, NCHUNK, unroll=4)
        def _(j):
            lab, key = load_chunk(j)
            isp = lab == 1
            isn = lab == 0
            if up_shift < 32:
                pm = key >> up_shift
                isp = isp & (pm == prefp)
                isn = isn & (pm == prefn)
            d = ((key >> shift) & (NB - 1)) + jnp.where(isn, jnp.int32(NB),
                                                        jnp.int32(0))
            plsc.addupdate_scatter(hist_v, [d], ones, mask=isp | isn)

        pltpu.sync_copy(hist_v, shrows.at[wid])
        plsc.subcore_barrier()

    def build_merged():
        pltpu.sync_copy(shrows.at[:, pl.ds(wid * SL, SL)], slice_v)

        @plsc.parallel_loop(0, SL // 16, unroll=2, carry=zeros16)
        def csv(i, csv):
            acc = zeros16
            for t in range(NT):
                acc = acc + slice_v[t, pl.ds(i * 16, 16)]
            mslice_v[pl.ds(i * 16, 16)] = acc
            return jnp.where(iota == i, jnp.sum(acc), csv)

        csums_v[...] = csv
        pltpu.sync_copy(mslice_v, shmerg.at[pl.ds(wid * SL, SL)])
        pltpu.sync_copy(csums_v, shcsum.at[wid])
        plsc.subcore_barrier()
        pltpu.sync_copy(shcsum, csall_v)

    def class_total(cls):
        tot = jnp.int32(0)
        for t in range(8):
            tot = tot + jnp.sum(csall_v[cls * 8 + t])
        return tot

    def dscan(cls, K_rem):
        def p1(i, carry):
            running, C, Gc = carry
            r = 7 - i
            v = csall_v[cls * 8 + r]
            s_in = lax.rev(jnp.cumsum(lax.rev(v, (0,))), (0,))
            S = s_in + running
            cidx = r * 16 + iota
            qual = S >= K_rem
            C = jnp.maximum(C, jnp.max(jnp.where(qual, cidx, -1)))
            Gc = jnp.minimum(Gc, jnp.min(jnp.where(qual, S - v, BIG)))
            running = running + jnp.sum(v)
            return running, C, Gc
        _, C, Gc = lax.fori_loop(0, 8, p1,
                                 (jnp.int32(0), jnp.int32(-1), BIG))
        Cc = jnp.maximum(C, 0)
        pltpu.sync_copy(shmerg.at[pl.ds(cls * NB + Cc * 16, 16)], tmp16_v)
        h = tmp16_v[...]
        s_in = lax.rev(jnp.cumsum(lax.rev(h, (0,))), (0,))
        S = s_in + Gc
        b_idx = Cc * 16 + iota
        qual = S >= K_rem
        D = jnp.max(jnp.where(qual, b_idx, -1))
        G = jnp.min(jnp.where(qual, S - h, BIG))
        return D, G

    hist_pass(32, 21, jnp.int32(0), jnp.int32(0))
    build_merged()

    num_pos = class_total(0)
    K_pos = jnp.minimum(num_pos, jnp.int32(MAXPOS))
    D1p, G = dscan(0, K_pos)
    Kp_rem = K_pos - G

    num_neg = class_total(1)
    K_neg = jnp.minimum(jnp.int32(BATCH) - K_pos, num_neg)
    D1n, G = dscan(1, K_neg)
    Kn_rem = K_neg - G

    hist_pass(21, 10, D1p, D1n)
    build_merged()
    D2p, G = dscan(0, Kp_rem)
    Kp_rem = Kp_rem - G
    D2n, G = dscan(1, Kn_rem)
    Kn_rem = Kn_rem - G

    pref2p = (D1p << 11) | D2p
    pref2n = (D1n << 11) | D2n
    hist_pass(10, 0, pref2p, pref2n)
    build_merged()
    D3p, G = dscan(0, Kp_rem)
    r_p = Kp_rem - G
    D3n, G = dscan(1, Kn_rem)
    r_n = Kn_rem - G

    e_p = jnp.max(plsc.load_gather(hist_v, [jnp.maximum(D3p, 0) + zeros16]))
    e_n = jnp.max(plsc.load_gather(
        hist_v, [jnp.maximum(D3n, 0) + NB + zeros16]))
    statsme_v[...] = jnp.where(iota == 0, e_p,
                               jnp.where(iota == 1, e_n, jnp.int32(0)))
    pltpu.sync_copy(statsme_v, sstats.at[wid])
    plsc.subcore_barrier()
    pltpu.sync_copy(sstats, stats_v)
    colp = plsc.load_gather(stats_v, [iota, zeros16])
    coln = plsc.load_gather(stats_v, [iota, zeros16 + 1])
    below = iota < wid
    base_p = jnp.sum(jnp.where(below, colp, jnp.int32(0)))
    base_n = jnp.sum(jnp.where(below, coln, jnp.int32(0)))

    T_p = (D1p << 21) | (D2p << 10) | D3p
    T_n = (D1n << 21) | (D2n << 10) | D3n
    maxkey = jnp.int32(0x7FFFFFFF)
    T_p = jnp.where(K_pos == 0, maxkey, T_p)
    r_p = jnp.where(K_pos == 0, jnp.int32(0), r_p)
    T_n = jnp.where(K_neg == 0, maxkey, T_n)
    r_n = jnp.where(K_neg == 0, jnp.int32(0), r_n)

    allp = (base_p + e_p) <= r_p
    alln = (base_n + e_n) <= r_n
    partp = (base_p < r_p) & jnp.logical_not(allp)
    partn = (base_n < r_n) & jnp.logical_not(alln)

    @plsc.parallel_loop(0, NCHUNK, unroll=2)
    def _(j):
        lab, key = load_chunk(j)
        selp = (lab == 1) & ((key > T_p) | ((key == T_p) & allp))
        seln = (lab == 0) & ((key > T_n) | ((key == T_n) & alln))
        out_v[pl.ds(j * 16, 16)] = (
            jnp.where(selp, jnp.int32(1), jnp.int32(0))
            | jnp.where(seln, jnp.int32(2), jnp.int32(0)))

    @pl.when(partp)
    def _():
        def fb(j, rk):
            lab, key = load_chunk(j)
            eq = (lab == 1) & (key == T_p)
            eqi = jnp.where(eq, jnp.int32(1), jnp.int32(0))
            excl = jnp.cumsum(eqi) - eqi
            add = eq & ((rk + excl) < r_p)
            cur = out_v[pl.ds(j * 16, 16)]
            out_v[pl.ds(j * 16, 16)] = cur | jnp.where(add, jnp.int32(1),
                                                       jnp.int32(0))
            return rk + jnp.sum(eqi)
        lax.fori_loop(0, NCHUNK, fb, base_p)

    @pl.when(partn)
    def _():
        def fb(j, rk):
            lab, key = load_chunk(j)
            eq = (lab == 0) & (key == T_n)
            eqi = jnp.where(eq, jnp.int32(1), jnp.int32(0))
            excl = jnp.cumsum(eqi) - eqi
            add = eq & ((rk + excl) < r_n)
            cur = out_v[pl.ds(j * 16, 16)]
            out_v[pl.ds(j * 16, 16)] = cur | jnp.where(add, jnp.int32(2),
                                                       jnp.int32(0))
            return rk + jnp.sum(eqi)
        lax.fori_loop(0, NCHUNK, fb, base_n)

    pltpu.sync_copy(out_v, out_hbm.at[pl.ds(base, CH)])


_mesh = plsc.VectorSubcoreMesh(core_axis_name="c", subcore_axis_name="s",
                               num_cores=1)

_ohem = functools.partial(
    pl.kernel,
    mesh=_mesh,
    compiler_params=pltpu.CompilerParams(needs_layout_passes=False),
    out_type=jax.ShapeDtypeStruct((N,), jnp.int32),
    scratch_types=[
        pltpu.VMEM((CH,), jnp.int32),
        pltpu.VMEM((CH,), jnp.float32),
        pltpu.VMEM((CH,), jnp.int32),
        pltpu.VMEM((NB2,), jnp.int32),
        pltpu.VMEM((NT, SL), jnp.int32),
        pltpu.VMEM((SL,), jnp.int32),
        pltpu.VMEM((16,), jnp.int32),
        pltpu.VMEM((NT, 16), jnp.int32),
        pltpu.VMEM((16,), jnp.int32),
        pltpu.VMEM((NT, 16), jnp.int32),
        pltpu.VMEM((16,), jnp.int32),
        pltpu.VMEM_SHARED((NT, NB2), jnp.int32),
        pltpu.VMEM_SHARED((NB2,), jnp.int32),
        pltpu.VMEM_SHARED((NT, 16), jnp.int32),
        pltpu.VMEM_SHARED((NT, 16), jnp.int32),
    ],
)(_ohem_body)


def kernel(labels, losses):
    labels = labels.astype(jnp.int32)
    packed = _ohem(labels, losses.astype(jnp.float32))
    return (packed & 1) != 0, (packed & 2) != 0
